# baseline (speedup 1.0000x reference)
import jax
import jax.numpy as jnp
from jax.experimental import pallas as pl

_EFF_R = 2.0 * float(int(0.01 * 304 + 2) / 304)
_GRIDS = [(56, 40), (28, 20), (14, 10), (7, 5)]
_CART_MAX = [2.0 * _EFF_R, 0.1, 0.2, 0.4]


def _mm_block(x_ref, w_ref, b_ref, o_ref):
    o_ref[...] = (
        jnp.dot(x_ref[...], w_ref[...], preferred_element_type=jnp.float32)
        + b_ref[...]
    )


def _edge_matmul(xin, W, b, block_e=8192):
    e0, din = xin.shape
    dout = W.shape[1]
    dinp = ((din + 127) // 128) * 128
    doutp = ((dout + 127) // 128) * 128
    epad = ((e0 + block_e - 1) // block_e) * block_e
    xp = jnp.pad(xin, ((0, epad - e0), (0, dinp - din)))
    wp = jnp.pad(W, ((0, dinp - din), (0, doutp - dout)))
    bp = jnp.pad(b, ((0, doutp - dout),)).reshape(1, doutp)
    out = pl.pallas_call(
        _mm_block,
        grid=(epad // block_e,),
        in_specs=[
            pl.BlockSpec((block_e, dinp), lambda i: (i, 0)),
            pl.BlockSpec((dinp, doutp), lambda i: (0, 0)),
            pl.BlockSpec((1, doutp), lambda i: (0, 0)),
        ],
        out_specs=pl.BlockSpec((block_e, doutp), lambda i: (i, 0)),
        out_shape=jax.ShapeDtypeStruct((epad, doutp), jnp.float32),
    )(xp, wp, bp)
    return out[:e0, :dout]


def _eattr(pos, src, dst, max_value):
    delta = pos[dst, :2] - pos[src, :2]
    return jnp.clip(delta / (2.0 * max_value) + 0.5, 0.0, 1.0)


def _conv(h, eattr, src, dst, W, b, num_nodes):
    msg = _edge_matmul(jnp.concatenate([h[src], eattr], axis=1), W, b)
    agg = jax.ops.segment_sum(msg, dst, num_segments=num_nodes)
    return jax.nn.relu(agg)


def _pool(x, pos, edge_index, nx, ny, aggr):
    vx = 1.0 / nx
    vy = 1.0 / ny
    ix = jnp.clip(jnp.floor(pos[:, 0] / vx).astype(jnp.int32), 0, nx - 1)
    iy = jnp.clip(jnp.floor(pos[:, 1] / vy).astype(jnp.int32), 0, ny - 1)
    cluster = iy * nx + ix
    C = nx * ny
    cnt = jax.ops.segment_sum(
        jnp.ones((pos.shape[0],), dtype=pos.dtype), cluster, num_segments=C
    )
    denom = jnp.maximum(cnt, 1.0)[:, None]
    new_pos = jax.ops.segment_sum(pos, cluster, num_segments=C) / denom
    if aggr == 'max':
        xm = jax.ops.segment_max(x, cluster, num_segments=C)
        new_x = jnp.where(jnp.isfinite(xm), xm, 0.0)
    else:
        new_x = jax.ops.segment_sum(x, cluster, num_segments=C) / denom
    return new_x, new_pos, cluster[edge_index]


def kernel(x, pos, edge_index, W1, b1, W2, b2, W3, b3, W4, b4, W5, b5):
    n = x.shape[0]
    src, dst = edge_index[0], edge_index[1]
    ea = _eattr(pos, src, dst, _EFF_R)
    h = jnp.concatenate([x, pos[:, :2]], axis=1)
    h = _conv(h, ea, src, dst, W1, b1, n)
    params = [(W2, b2), (W3, b3), (W4, b4), (W5, b5)]
    aggrs = ['max', 'max', 'max', 'mean']
    cur_pos = pos
    ei = edge_index
    out3 = None
    for i in range(4):
        nx_, ny_ = _GRIDS[i]
        h, cur_pos, ei = _pool(h, cur_pos, ei, nx_, ny_, aggrs[i])
        src, dst = ei[0], ei[1]
        ea = _eattr(cur_pos, src, dst, _CART_MAX[i])
        h = jnp.concatenate([h, cur_pos[:, :2]], axis=1)
        h = _conv(h, ea, src, dst, params[i][0], params[i][1], nx_ * ny_)
        if i == 2:
            out3 = h
    return (out3, h)


# speedup vs baseline: 4.5623x; 4.5623x over previous
import functools
import jax
import jax.numpy as jnp
from jax import lax
from jax.experimental import pallas as pl
from jax.experimental.pallas import tpu as pltpu
from jax.experimental.pallas import tpu_sc as plsc

_EFF_R = 2.0 * float(int(0.01 * 304 + 2) / 304)
_GRIDS = [(56, 40), (28, 20), (14, 10), (7, 5)]
_CART_MAX = [2.0 * _EFF_R, 0.1, 0.2, 0.4]

_NC, _NS = 2, 16
_NW = _NC * _NS
_B = 128
_E = 800000
_CH = -(-_E // (_NW * _B))
_EPAD = _NW * _B * _CH


def _edge_accum(stab, dtab, sidx, didx, oidx, cvec, lovec, hivec, m_out):
    w = stab.shape[1]
    ng = w // 16
    z = -(-(m_out + 1) // (_NS * _B)) * _B
    mpad = _NS * z
    mesh = plsc.VectorSubcoreMesh(core_axis_name="c", subcore_axis_name="s")

    @functools.partial(
        pl.kernel,
        mesh=mesh,
        compiler_params=pltpu.CompilerParams(use_tc_tiling_on_sc=False),
        out_type=jax.ShapeDtypeStruct((_NC, mpad, w), jnp.float32),
        scratch_types=[
            pltpu.VMEM((_B,), jnp.int32),
            pltpu.VMEM((_B,), jnp.int32),
            pltpu.VMEM((_B,), jnp.int32),
            pltpu.VMEM((_B, w), jnp.float32),
            pltpu.VMEM((_B, w), jnp.float32),
            pltpu.VMEM((_B, w), jnp.float32),
            pltpu.VMEM((_B, w), jnp.float32),
            pltpu.VMEM((w,), jnp.float32),
            pltpu.VMEM((w,), jnp.float32),
            pltpu.VMEM((w,), jnp.float32),
            pltpu.VMEM_SHARED((mpad, w), jnp.float32),
            pltpu.SemaphoreType.DMA,
            pltpu.SemaphoreType.DMA,
        ],
    )
    def k(stab_h, dtab_h, sidx_h, didx_h, oidx_h, cvec_h, lovec_h, hivec_h,
          zrows_h, out_h, sidx_v, didx_v, oidx_v, srows_v, drows_v, mrows_v,
          zrows_v, cvec_v, lovec_v, hivec_v, acc_s, sem1, sem2):
        cid = lax.axis_index("c")
        sid = lax.axis_index("s")
        wid = sid * _NC + cid
        pltpu.sync_copy(cvec_h, cvec_v)
        pltpu.sync_copy(lovec_h, lovec_v)
        pltpu.sync_copy(hivec_h, hivec_v)
        pltpu.sync_copy(zrows_h, zrows_v)
        for t in range(z // _B):
            pltpu.sync_copy(zrows_v, acc_s.at[pl.ds(sid * z + t * _B, _B)])
        plsc.subcore_barrier()

        cs = [cvec_v[pl.ds(16 * g, 16)] for g in range(ng)]
        los = [lovec_v[pl.ds(16 * g, 16)] for g in range(ng)]
        his = [hivec_v[pl.ds(16 * g, 16)] for g in range(ng)]

        def chunk_body(t, carry):
            base = (wid * _CH + t) * _B
            pltpu.sync_copy(sidx_h.at[pl.ds(base, _B)], sidx_v)
            pltpu.sync_copy(didx_h.at[pl.ds(base, _B)], didx_v)
            pltpu.sync_copy(oidx_h.at[pl.ds(base, _B)], oidx_v)
            c1 = pltpu.async_copy(stab_h.at[sidx_v], srows_v, sem1)
            c2 = pltpu.async_copy(dtab_h.at[didx_v], drows_v, sem2)
            c1.wait()
            c2.wait()

            def row_body(j, carry2):
                for g in range(ng):
                    s = srows_v[j, pl.ds(16 * g, 16)]
                    d = drows_v[j, pl.ds(16 * g, 16)]
                    m = jnp.minimum(jnp.maximum(d - s + cs[g], los[g]), his[g])
                    mrows_v[j, pl.ds(16 * g, 16)] = m
                return carry2

            lax.fori_loop(0, _B, row_body, 0)
            pltpu.sync_copy(mrows_v, acc_s.at[oidx_v], add=True)
            return carry

        lax.fori_loop(0, _CH, chunk_body, 0)
        plsc.subcore_barrier()
        for t in range(z // _B):
            r = sid * z + t * _B
            pltpu.sync_copy(acc_s.at[pl.ds(r, _B)], out_h.at[cid, pl.ds(r, _B)])

    out = k(stab, dtab, sidx, didx, oidx, cvec, lovec, hivec,
            jnp.zeros((_B, w), jnp.float32))
    return out[0] + out[1]


def _mm_relu_block(x_ref, w_ref, o_ref):
    o_ref[...] = jax.nn.relu(
        jnp.dot(x_ref[...], w_ref[...], preferred_element_type=jnp.float32)
    )


def _node_mm(xin, W, b, block_r=2048):
    din, dout = W.shape
    x = xin[:, : din + 1]
    wfull = jnp.concatenate([W, b[None, :]], axis=0)
    r0 = x.shape[0]
    dinp = ((din + 1 + 127) // 128) * 128
    doutp = ((dout + 127) // 128) * 128
    rpad = ((r0 + block_r - 1) // block_r) * block_r
    xp = jnp.pad(x, ((0, rpad - r0), (0, dinp - din - 1)))
    wp = jnp.pad(wfull, ((0, dinp - din - 1), (0, doutp - dout)))
    out = pl.pallas_call(
        _mm_relu_block,
        grid=(rpad // block_r,),
        in_specs=[
            pl.BlockSpec((block_r, dinp), lambda i: (i, 0)),
            pl.BlockSpec((dinp, doutp), lambda i: (0, 0)),
        ],
        out_specs=pl.BlockSpec((block_r, doutp), lambda i: (i, 0)),
        out_shape=jax.ShapeDtypeStruct((rpad, doutp), jnp.float32),
    )(xp, wp)
    return out[:r0, :dout]


def _tables(g, p2, scale, w):
    n, f = g.shape
    sp = p2 * scale
    pad = jnp.zeros((n, w - f - 3), jnp.float32)
    one = jnp.ones((n, 1), jnp.float32)
    zf = jnp.zeros((n, f), jnp.float32)
    z1 = jnp.zeros((n, 1), jnp.float32)
    stab = jnp.concatenate([-g, sp, z1, pad], axis=1)
    dtab = jnp.concatenate([zf, sp, one, pad], axis=1)
    zrow = jnp.zeros((1, w), jnp.float32)
    stab = jnp.concatenate([stab, zrow], axis=0)
    dtab = jnp.concatenate([dtab, zrow], axis=0)
    cvec = jnp.zeros((w,), jnp.float32).at[f:f + 2].set(0.5)
    lovec = jnp.full((w,), -1e30, jnp.float32).at[f:f + 2].set(0.0)
    hivec = jnp.full((w,), 1e30, jnp.float32).at[f:f + 2].set(1.0)
    return stab, dtab, cvec, lovec, hivec


def _pad_idx(a, fill):
    return jnp.concatenate(
        [a, jnp.full((_EPAD - _E,), fill, jnp.int32)]
    )


def _conv_level(g, p2, scale, src_l, dst_l, W, b, c_nodes):
    f = g.shape[1]
    w = ((f + 3 + 15) // 16) * 16
    stab, dtab, cv, lo, hi = _tables(g, p2, scale, w)
    sidx = _pad_idx(src_l, c_nodes)
    didx = _pad_idx(dst_l, c_nodes)
    oidx = _pad_idx(dst_l, c_nodes)
    acc = _edge_accum(stab, dtab, sidx, didx, oidx, cv, lo, hi, c_nodes)
    return _node_mm(acc[:c_nodes], W, b)


def _pool(x, pos, nx, ny, aggr):
    vx = 1.0 / nx
    vy = 1.0 / ny
    ix = jnp.clip(jnp.floor(pos[:, 0] / vx).astype(jnp.int32), 0, nx - 1)
    iy = jnp.clip(jnp.floor(pos[:, 1] / vy).astype(jnp.int32), 0, ny - 1)
    cluster = iy * nx + ix
    C = nx * ny
    cnt = jax.ops.segment_sum(
        jnp.ones((pos.shape[0],), dtype=pos.dtype), cluster, num_segments=C
    )
    denom = jnp.maximum(cnt, 1.0)[:, None]
    new_pos = jax.ops.segment_sum(pos, cluster, num_segments=C) / denom
    if aggr == 'max':
        xm = jax.ops.segment_max(x, cluster, num_segments=C)
        new_x = jnp.where(jnp.isfinite(xm), xm, 0.0)
    else:
        new_x = jax.ops.segment_sum(x, cluster, num_segments=C) / denom
    return new_x, new_pos, cluster


def kernel(x, pos, edge_index, W1, b1, W2, b2, W3, b3, W4, b4, W5, b5):
    n = x.shape[0]
    src, dst = edge_index[0], edge_index[1]
    g1 = jnp.concatenate([x, pos[:, :2]], axis=1)
    h = _conv_level(g1, pos[:, :2], 1.0 / (2.0 * _EFF_R), src, dst, W1, b1, n)
    params = [(W2, b2), (W3, b3), (W4, b4), (W5, b5)]
    aggrs = ['max', 'max', 'max', 'mean']
    cur_pos = pos
    node_map = None
    out3 = None
    for i in range(4):
        nx_, ny_ = _GRIDS[i]
        h, cur_pos, cl = _pool(h, cur_pos, nx_, ny_, aggrs[i])
        node_map = cl if node_map is None else cl[node_map]
        g = jnp.concatenate([h, cur_pos[:, :2]], axis=1)
        h = _conv_level(
            g, cur_pos[:, :2], 1.0 / (2.0 * _CART_MAX[i]),
            node_map[src], node_map[dst],
            params[i][0], params[i][1], nx_ * ny_,
        )
        if i == 2:
            out3 = h
    return (out3, h)
